# baseline (speedup 1.0000x reference)
import functools

import jax
import jax.numpy as jnp
from jax import lax
from jax.experimental import pallas as pl
from jax.experimental.pallas import tpu as pltpu
from jax.experimental.pallas import tpu_sc as plsc

_ROWS = 6
_D = 64
_L = 16

_mesh = plsc.VectorSubcoreMesh(core_axis_name="c", subcore_axis_name="s")


@functools.partial(
    pl.kernel,
    out_type=jax.ShapeDtypeStruct((_ROWS, _D), jnp.float32),
    mesh=_mesh,
    scratch_types=[
        pltpu.VMEM((_ROWS + 2, _D), jnp.float32),
        pltpu.VMEM((_ROWS, _D), jnp.float32),
    ],
)
def _shifted_add(x_hbm, out_hbm, xbuf, obuf):
    cid = lax.axis_index("c")
    sid = lax.axis_index("s")

    @pl.when((cid == 0) & (sid == 0))
    def _():
        pltpu.sync_copy(x_hbm.at[pl.ds(0, _ROWS + 2)], xbuf)
        for r in range(_ROWS):
            for c in range(0, _D, _L):
                obuf[r, pl.ds(c, _L)] = (
                    xbuf[r, pl.ds(c, _L)] + xbuf[r + 1, pl.ds(c, _L)]
                )
        pltpu.sync_copy(obuf, out_hbm)


def kernel(x):
    return _shifted_add(x).reshape(3, 2, _D)


# speedup vs baseline: 1.0060x; 1.0060x over previous
import functools

import jax
import jax.numpy as jnp
from jax import lax
from jax.experimental import pallas as pl
from jax.experimental.pallas import tpu as pltpu
from jax.experimental.pallas import tpu_sc as plsc

_ROWS = 6
_D = 64
_L = 16

_mesh = plsc.VectorSubcoreMesh(
    core_axis_name="c", subcore_axis_name="s", num_cores=1, num_subcores=1
)


@functools.partial(
    pl.kernel,
    out_type=jax.ShapeDtypeStruct((_ROWS, _D), jnp.float32),
    mesh=_mesh,
    scratch_types=[
        pltpu.VMEM((_ROWS + 2, _D), jnp.float32),
        pltpu.VMEM((_ROWS, _D), jnp.float32),
    ],
)
def _shifted_add(x_hbm, out_hbm, xbuf, obuf):
    cid = lax.axis_index("c")
    sid = lax.axis_index("s")

    @pl.when((cid == 0) & (sid == 0))
    def _():
        pltpu.sync_copy(x_hbm.at[pl.ds(0, _ROWS + 2)], xbuf)
        for r in range(_ROWS):
            for c in range(0, _D, _L):
                obuf[r, pl.ds(c, _L)] = (
                    xbuf[r, pl.ds(c, _L)] + xbuf[r + 1, pl.ds(c, _L)]
                )
        pltpu.sync_copy(obuf, out_hbm)


def kernel(x):
    return _shifted_add(x).reshape(3, 2, _D)


# speedup vs baseline: 18.0954x; 17.9870x over previous
import functools

import jax
import jax.numpy as jnp
from jax import lax
from jax.experimental import pallas as pl
from jax.experimental.pallas import tpu as pltpu
from jax.experimental.pallas import tpu_sc as plsc

_ROWS = 6
_D = 64
_L = 16

_mesh = plsc.VectorSubcoreMesh(
    core_axis_name="c", subcore_axis_name="s", num_cores=1, num_subcores=1
)


@functools.partial(
    pl.kernel,
    out_type=jax.ShapeDtypeStruct((_ROWS, _D), jnp.float32),
    mesh=_mesh,
    scratch_types=[
        pltpu.VMEM((_ROWS + 2, _D), jnp.float32),
        pltpu.VMEM((_ROWS, _D), jnp.float32),
    ],
)
def _shifted_add(x_hbm, out_hbm, xbuf, obuf):
    cid = lax.axis_index("c")
    sid = lax.axis_index("s")

    @pl.when((cid == 0) & (sid == 0))
    def _():
        pltpu.sync_copy(x_hbm, xbuf)
        for r in range(_ROWS):
            for c in range(0, _D, _L):
                obuf[r, pl.ds(c, _L)] = (
                    xbuf[r, pl.ds(c, _L)] + xbuf[r + 1, pl.ds(c, _L)]
                )
        pltpu.sync_copy(obuf, out_hbm)


def kernel(x):
    head = jax.lax.slice(x, (0, 0), (_ROWS + 2, _D))
    return _shifted_add(head).reshape(3, 2, _D)
